# Initial kernel scaffold; baseline (speedup 1.0000x reference)
import jax
import jax.numpy as jnp
from jax.experimental import pallas as pl


def kernel(logits, actions):
    raise NotImplementedError("write your pallas kernel here")


# speedup vs baseline: 2.7016x; 2.7016x over previous
import functools

import jax
import jax.numpy as jnp
from jax import lax
from jax.experimental import pallas as pl
from jax.experimental.pallas import tpu as pltpu

B = 32
V = 1000000
CB = 16384
NB = (V + CB - 1) // CB


def _body(x_ref, a_ref, lp_ref, mode_ref, m_ref, s_ref, g_ref, idx_ref):
    j = pl.program_id(0)

    @pl.when(j == 0)
    def _init():
        m_ref[...] = jnp.full((B, 1), -jnp.inf, jnp.float32)
        s_ref[...] = jnp.zeros((B, 1), jnp.float32)
        g_ref[...] = jnp.zeros((B, 1), jnp.float32)
        idx_ref[...] = jnp.zeros((B, 1), jnp.int32)

    a = a_ref[...]
    col = lax.broadcasted_iota(jnp.int32, (B, CB), 1) + j * CB

    def process(x):
        bmax = jnp.max(x, axis=1, keepdims=True)
        bsum = jnp.sum(jnp.exp(x - bmax), axis=1, keepdims=True)
        cand = jnp.where(x == bmax, col, jnp.int32(V))
        bidx = jnp.min(cand, axis=1, keepdims=True)
        gq = jnp.sum(jnp.where(col == a, x, 0.0), axis=1, keepdims=True)
        m = m_ref[...]
        better = bmax > m
        mnew = jnp.where(better, bmax, m)
        s_ref[...] = s_ref[...] * jnp.exp(m - mnew) + bsum * jnp.exp(bmax - mnew)
        idx_ref[...] = jnp.where(better, bidx, idx_ref[...])
        m_ref[...] = mnew
        g_ref[...] = g_ref[...] + gq

    @pl.when(j < NB - 1)
    def _full():
        process(x_ref[...])

    @pl.when(j == NB - 1)
    def _partial():
        x = jnp.where(col < V, x_ref[...], -jnp.inf)
        process(x)
        lp_ref[...] = g_ref[...] - m_ref[...] - jnp.log(s_ref[...])
        mode_ref[...] = idx_ref[...]


@jax.jit
def _run(logits, actions):
    lp, mode = pl.pallas_call(
        _body,
        grid=(NB,),
        in_specs=[
            pl.BlockSpec((B, CB), lambda j: (0, j)),
            pl.BlockSpec((B, 1), lambda j: (0, 0)),
        ],
        out_specs=[
            pl.BlockSpec((B, 1), lambda j: (0, 0)),
            pl.BlockSpec((B, 1), lambda j: (0, 0)),
        ],
        out_shape=[
            jax.ShapeDtypeStruct((B, 1), jnp.float32),
            jax.ShapeDtypeStruct((B, 1), jnp.int32),
        ],
        scratch_shapes=[
            pltpu.VMEM((B, 1), jnp.float32),
            pltpu.VMEM((B, 1), jnp.float32),
            pltpu.VMEM((B, 1), jnp.float32),
            pltpu.VMEM((B, 1), jnp.int32),
        ],
        compiler_params=pltpu.CompilerParams(
            dimension_semantics=("arbitrary",),
        ),
    )(logits, actions)
    return lp, mode


def kernel(logits, actions):
    a = actions.astype(jnp.int32).reshape(B, 1)
    return _run(logits, a)


# speedup vs baseline: 2.8954x; 1.0717x over previous
import jax
import jax.numpy as jnp
from jax import lax
from jax.experimental import pallas as pl
from jax.experimental.pallas import tpu as pltpu

B = 32
V = 1000000
CB = 16384
NB = (V + CB - 1) // CB


def _stream_body(x_ref, a_ref, lp0_ref, m_ref, blk_ref, it_ref, gt_ref,
                 s_ref):
    j = pl.program_id(0)

    @pl.when(j == 0)
    def _init():
        m_ref[...] = jnp.full((B, 1), -jnp.inf, jnp.float32)
        blk_ref[...] = jnp.zeros((B, 1), jnp.int32)
        s_ref[...] = jnp.zeros((B, 1), jnp.float32)

    def process(x):
        bmax = jnp.max(x, axis=1, keepdims=True)
        bsum = jnp.sum(jnp.exp(x - bmax), axis=1, keepdims=True)
        m = m_ref[...]
        mnew = jnp.maximum(m, bmax)
        s_ref[...] = s_ref[...] * jnp.exp(m - mnew) + bsum * jnp.exp(bmax - mnew)
        blk_ref[...] = jnp.where(bmax > m, j, blk_ref[...])
        m_ref[...] = mnew
        return bmax

    @pl.when(j < NB - 1)
    def _full():
        process(x_ref[...])

    @pl.when(j == NB - 1)
    def _partial():
        col = lax.broadcasted_iota(jnp.int32, (B, CB), 1) + j * CB
        x = jnp.where(col < V, x_ref[...], -jnp.inf)
        bmax = process(x)
        cand = jnp.where(x == bmax, col, jnp.int32(V))
        it_ref[...] = jnp.min(cand, axis=1, keepdims=True)
        gt_ref[...] = jnp.sum(jnp.where(col == a_ref[...], x, 0.0), axis=1,
                              keepdims=True)
        lp0_ref[...] = -m_ref[...] - jnp.log(s_ref[...])


def _recover_body(blk_s, ablk_s, hbm_ref, m_ref, a_ref, lp0_ref, blkv_ref,
                  ablkv_ref, it_ref, gt_ref, lp_ref, mode_ref,
                  xm_scr, xa_scr, sem):
    copies = []
    for i in range(B):
        o1 = jnp.minimum(blk_s[i], NB - 2) * CB
        c1 = pltpu.make_async_copy(
            hbm_ref.at[pl.ds(i, 1), pl.ds(o1, CB)],
            xm_scr.at[pl.ds(i, 1), :], sem)
        c1.start()
        copies.append(c1)
        o2 = jnp.minimum(ablk_s[i], NB - 2) * CB
        c2 = pltpu.make_async_copy(
            hbm_ref.at[pl.ds(i, 1), pl.ds(o2, CB)],
            xa_scr.at[pl.ds(i, 1), :], sem)
        c2.start()
        copies.append(c2)
    for c in copies:
        c.wait()

    m = m_ref[...]
    a = a_ref[...]
    blkv = blkv_ref[...]
    ablkv = ablkv_ref[...]
    last = jnp.int32(NB - 1)

    col_m = (lax.broadcasted_iota(jnp.int32, (B, CB), 1)
             + jnp.minimum(blkv, NB - 2) * CB)
    cand = jnp.where(xm_scr[...] == m, col_m, jnp.int32(V))
    idx = jnp.min(cand, axis=1, keepdims=True)
    idx = jnp.where(blkv == last, it_ref[...], idx)

    col_a = (lax.broadcasted_iota(jnp.int32, (B, CB), 1)
             + jnp.minimum(ablkv, NB - 2) * CB)
    g = jnp.sum(jnp.where(col_a == a, xa_scr[...], 0.0), axis=1,
                keepdims=True)
    g = jnp.where(ablkv == last, gt_ref[...], g)

    lp_ref[...] = g + lp0_ref[...]
    mode_ref[...] = idx


def _build(interpret=False):
    stream = pl.pallas_call(
        _stream_body,
        grid=(NB,),
        in_specs=[pl.BlockSpec((B, CB), lambda j: (0, j)),
                  pl.BlockSpec((B, 1), lambda j: (0, 0))],
        out_specs=[pl.BlockSpec((B, 1), lambda j: (0, 0)),
                   pl.BlockSpec((B, 1), lambda j: (0, 0)),
                   pl.BlockSpec((B, 1), lambda j: (0, 0)),
                   pl.BlockSpec((B, 1), lambda j: (0, 0)),
                   pl.BlockSpec((B, 1), lambda j: (0, 0))],
        out_shape=[jax.ShapeDtypeStruct((B, 1), jnp.float32),
                   jax.ShapeDtypeStruct((B, 1), jnp.float32),
                   jax.ShapeDtypeStruct((B, 1), jnp.int32),
                   jax.ShapeDtypeStruct((B, 1), jnp.int32),
                   jax.ShapeDtypeStruct((B, 1), jnp.float32)],
        scratch_shapes=[pltpu.VMEM((B, 1), jnp.float32)],
        compiler_params=pltpu.CompilerParams(
            dimension_semantics=("arbitrary",)),
        interpret=interpret,
    )

    recover = pl.pallas_call(
        _recover_body,
        grid_spec=pltpu.PrefetchScalarGridSpec(
            num_scalar_prefetch=2,
            grid=(1,),
            in_specs=[
                pl.BlockSpec(memory_space=pl.ANY),
                pl.BlockSpec((B, 1), lambda i, blk, ablk: (0, 0)),
                pl.BlockSpec((B, 1), lambda i, blk, ablk: (0, 0)),
                pl.BlockSpec((B, 1), lambda i, blk, ablk: (0, 0)),
                pl.BlockSpec((B, 1), lambda i, blk, ablk: (0, 0)),
                pl.BlockSpec((B, 1), lambda i, blk, ablk: (0, 0)),
                pl.BlockSpec((B, 1), lambda i, blk, ablk: (0, 0)),
                pl.BlockSpec((B, 1), lambda i, blk, ablk: (0, 0)),
            ],
            out_specs=[pl.BlockSpec((B, 1), lambda i, blk, ablk: (0, 0)),
                       pl.BlockSpec((B, 1), lambda i, blk, ablk: (0, 0))],
            scratch_shapes=[pltpu.VMEM((B, CB), jnp.float32),
                            pltpu.VMEM((B, CB), jnp.float32),
                            pltpu.SemaphoreType.DMA],
        ),
        out_shape=[jax.ShapeDtypeStruct((B, 1), jnp.float32),
                   jax.ShapeDtypeStruct((B, 1), jnp.int32)],
        interpret=interpret,
    )

    @jax.jit
    def run(logits, actions):
        a = actions.astype(jnp.int32).reshape(B, 1)
        lp0, m, blk, it, gt = stream(logits, a)
        ablk = a // CB
        lp, mode = recover(blk.reshape(B), ablk.reshape(B), logits, m, a,
                           lp0, blk, ablk, it, gt)
        return lp, mode

    return run


_run = _build()


def kernel(logits, actions):
    return _run(logits, actions)


# speedup vs baseline: 3.5329x; 1.2202x over previous
import jax
import jax.numpy as jnp
from jax import lax
from jax.experimental import pallas as pl
from jax.experimental.pallas import tpu as pltpu

B = 32
V = 1000000
CB = 32768
NB = (V + CB - 1) // CB


def _stream_body(x_ref, a_ref, lp0_ref, m_ref, blk_ref, it_ref, gt_ref,
                 s_ref):
    j = pl.program_id(0)

    @pl.when(j == 0)
    def _init():
        m_ref[...] = jnp.full((B, 1), -jnp.inf, jnp.float32)
        blk_ref[...] = jnp.zeros((B, 1), jnp.int32)
        s_ref[...] = jnp.zeros((B, 1), jnp.float32)

    def process(x):
        bmax = jnp.max(x, axis=1, keepdims=True)
        bsum = jnp.sum(jnp.exp(x - bmax), axis=1, keepdims=True)
        m = m_ref[...]
        mnew = jnp.maximum(m, bmax)
        s_ref[...] = s_ref[...] * jnp.exp(m - mnew) + bsum * jnp.exp(bmax - mnew)
        blk_ref[...] = jnp.where(bmax > m, j, blk_ref[...])
        m_ref[...] = mnew
        return bmax

    @pl.when(j < NB - 1)
    def _full():
        process(x_ref[...])

    @pl.when(j == NB - 1)
    def _partial():
        col = lax.broadcasted_iota(jnp.int32, (B, CB), 1) + j * CB
        x = jnp.where(col < V, x_ref[...], -jnp.inf)
        bmax = process(x)
        cand = jnp.where(x == bmax, col, jnp.int32(V))
        it_ref[...] = jnp.min(cand, axis=1, keepdims=True)
        gt_ref[...] = jnp.sum(jnp.where(col == a_ref[...], x, 0.0), axis=1,
                              keepdims=True)
        lp0_ref[...] = -m_ref[...] - jnp.log(s_ref[...])


def _recover_body(blk_s, ablk_s, hbm_ref, m_ref, a_ref, lp0_ref, blkv_ref,
                  ablkv_ref, it_ref, gt_ref, lp_ref, mode_ref,
                  xm_scr, xa_scr, sem):
    copies = []
    for i in range(B):
        o1 = jnp.minimum(blk_s[i], NB - 2) * CB
        c1 = pltpu.make_async_copy(
            hbm_ref.at[pl.ds(i, 1), pl.ds(o1, CB)],
            xm_scr.at[pl.ds(i, 1), :], sem)
        c1.start()
        copies.append(c1)
        o2 = jnp.minimum(ablk_s[i], NB - 2) * CB
        c2 = pltpu.make_async_copy(
            hbm_ref.at[pl.ds(i, 1), pl.ds(o2, CB)],
            xa_scr.at[pl.ds(i, 1), :], sem)
        c2.start()
        copies.append(c2)
    for c in copies:
        c.wait()

    m = m_ref[...]
    a = a_ref[...]
    blkv = blkv_ref[...]
    ablkv = ablkv_ref[...]
    last = jnp.int32(NB - 1)

    col_m = (lax.broadcasted_iota(jnp.int32, (B, CB), 1)
             + jnp.minimum(blkv, NB - 2) * CB)
    cand = jnp.where(xm_scr[...] == m, col_m, jnp.int32(V))
    idx = jnp.min(cand, axis=1, keepdims=True)
    idx = jnp.where(blkv == last, it_ref[...], idx)

    col_a = (lax.broadcasted_iota(jnp.int32, (B, CB), 1)
             + jnp.minimum(ablkv, NB - 2) * CB)
    g = jnp.sum(jnp.where(col_a == a, xa_scr[...], 0.0), axis=1,
                keepdims=True)
    g = jnp.where(ablkv == last, gt_ref[...], g)

    lp_ref[...] = g + lp0_ref[...]
    mode_ref[...] = idx


def _build(interpret=False):
    stream = pl.pallas_call(
        _stream_body,
        grid=(NB,),
        in_specs=[pl.BlockSpec((B, CB), lambda j: (0, j)),
                  pl.BlockSpec((B, 1), lambda j: (0, 0))],
        out_specs=[pl.BlockSpec((B, 1), lambda j: (0, 0)),
                   pl.BlockSpec((B, 1), lambda j: (0, 0)),
                   pl.BlockSpec((B, 1), lambda j: (0, 0)),
                   pl.BlockSpec((B, 1), lambda j: (0, 0)),
                   pl.BlockSpec((B, 1), lambda j: (0, 0))],
        out_shape=[jax.ShapeDtypeStruct((B, 1), jnp.float32),
                   jax.ShapeDtypeStruct((B, 1), jnp.float32),
                   jax.ShapeDtypeStruct((B, 1), jnp.int32),
                   jax.ShapeDtypeStruct((B, 1), jnp.int32),
                   jax.ShapeDtypeStruct((B, 1), jnp.float32)],
        scratch_shapes=[pltpu.VMEM((B, 1), jnp.float32)],
        compiler_params=pltpu.CompilerParams(
            dimension_semantics=("arbitrary",)),
        interpret=interpret,
    )

    recover = pl.pallas_call(
        _recover_body,
        grid_spec=pltpu.PrefetchScalarGridSpec(
            num_scalar_prefetch=2,
            grid=(1,),
            in_specs=[
                pl.BlockSpec(memory_space=pl.ANY),
                pl.BlockSpec((B, 1), lambda i, blk, ablk: (0, 0)),
                pl.BlockSpec((B, 1), lambda i, blk, ablk: (0, 0)),
                pl.BlockSpec((B, 1), lambda i, blk, ablk: (0, 0)),
                pl.BlockSpec((B, 1), lambda i, blk, ablk: (0, 0)),
                pl.BlockSpec((B, 1), lambda i, blk, ablk: (0, 0)),
                pl.BlockSpec((B, 1), lambda i, blk, ablk: (0, 0)),
                pl.BlockSpec((B, 1), lambda i, blk, ablk: (0, 0)),
            ],
            out_specs=[pl.BlockSpec((B, 1), lambda i, blk, ablk: (0, 0)),
                       pl.BlockSpec((B, 1), lambda i, blk, ablk: (0, 0))],
            scratch_shapes=[pltpu.VMEM((B, CB), jnp.float32),
                            pltpu.VMEM((B, CB), jnp.float32),
                            pltpu.SemaphoreType.DMA],
        ),
        out_shape=[jax.ShapeDtypeStruct((B, 1), jnp.float32),
                   jax.ShapeDtypeStruct((B, 1), jnp.int32)],
        interpret=interpret,
    )

    @jax.jit
    def run(logits, actions):
        a = actions.astype(jnp.int32).reshape(B, 1)
        lp0, m, blk, it, gt = stream(logits, a)
        ablk = a // CB
        lp, mode = recover(blk.reshape(B), ablk.reshape(B), logits, m, a,
                           lp0, blk, ablk, it, gt)
        return lp, mode

    return run


_run = _build()


def kernel(logits, actions):
    return _run(logits, actions)


# speedup vs baseline: 3.6614x; 1.0364x over previous
import jax
import jax.numpy as jnp
from jax import lax
from jax.experimental import pallas as pl
from jax.experimental.pallas import tpu as pltpu

B = 32
V = 1000000
CB = 65536
NB = (V + CB - 1) // CB


def _stream_body(x_ref, a_ref, lp0_ref, m_ref, blk_ref, it_ref, gt_ref,
                 s_ref):
    j = pl.program_id(0)

    @pl.when(j == 0)
    def _init():
        m_ref[...] = jnp.full((B, 1), -jnp.inf, jnp.float32)
        blk_ref[...] = jnp.zeros((B, 1), jnp.int32)
        s_ref[...] = jnp.zeros((B, 1), jnp.float32)

    def process(x):
        bmax = jnp.max(x, axis=1, keepdims=True)
        bsum = jnp.sum(jnp.exp(x - bmax), axis=1, keepdims=True)
        m = m_ref[...]
        mnew = jnp.maximum(m, bmax)
        s_ref[...] = s_ref[...] * jnp.exp(m - mnew) + bsum * jnp.exp(bmax - mnew)
        blk_ref[...] = jnp.where(bmax > m, j, blk_ref[...])
        m_ref[...] = mnew
        return bmax

    @pl.when(j < NB - 1)
    def _full():
        process(x_ref[...])

    @pl.when(j == NB - 1)
    def _partial():
        col = lax.broadcasted_iota(jnp.int32, (B, CB), 1) + j * CB
        x = jnp.where(col < V, x_ref[...], -jnp.inf)
        bmax = process(x)
        cand = jnp.where(x == bmax, col, jnp.int32(V))
        it_ref[...] = jnp.min(cand, axis=1, keepdims=True)
        gt_ref[...] = jnp.sum(jnp.where(col == a_ref[...], x, 0.0), axis=1,
                              keepdims=True)
        lp0_ref[...] = -m_ref[...] - jnp.log(s_ref[...])


def _recover_body(blk_s, ablk_s, hbm_ref, m_ref, a_ref, lp0_ref, blkv_ref,
                  ablkv_ref, it_ref, gt_ref, lp_ref, mode_ref,
                  xm_scr, xa_scr, sem):
    copies = []
    for i in range(B):
        o1 = jnp.minimum(blk_s[i], NB - 2) * CB
        c1 = pltpu.make_async_copy(
            hbm_ref.at[pl.ds(i, 1), pl.ds(o1, CB)],
            xm_scr.at[pl.ds(i, 1), :], sem)
        c1.start()
        copies.append(c1)
        o2 = jnp.minimum(ablk_s[i], NB - 2) * CB
        c2 = pltpu.make_async_copy(
            hbm_ref.at[pl.ds(i, 1), pl.ds(o2, CB)],
            xa_scr.at[pl.ds(i, 1), :], sem)
        c2.start()
        copies.append(c2)
    for c in copies:
        c.wait()

    m = m_ref[...]
    a = a_ref[...]
    blkv = blkv_ref[...]
    ablkv = ablkv_ref[...]
    last = jnp.int32(NB - 1)

    col_m = (lax.broadcasted_iota(jnp.int32, (B, CB), 1)
             + jnp.minimum(blkv, NB - 2) * CB)
    cand = jnp.where(xm_scr[...] == m, col_m, jnp.int32(V))
    idx = jnp.min(cand, axis=1, keepdims=True)
    idx = jnp.where(blkv == last, it_ref[...], idx)

    col_a = (lax.broadcasted_iota(jnp.int32, (B, CB), 1)
             + jnp.minimum(ablkv, NB - 2) * CB)
    g = jnp.sum(jnp.where(col_a == a, xa_scr[...], 0.0), axis=1,
                keepdims=True)
    g = jnp.where(ablkv == last, gt_ref[...], g)

    lp_ref[...] = g + lp0_ref[...]
    mode_ref[...] = idx


def _build(interpret=False):
    stream = pl.pallas_call(
        _stream_body,
        grid=(NB,),
        in_specs=[pl.BlockSpec((B, CB), lambda j: (0, j)),
                  pl.BlockSpec((B, 1), lambda j: (0, 0))],
        out_specs=[pl.BlockSpec((B, 1), lambda j: (0, 0)),
                   pl.BlockSpec((B, 1), lambda j: (0, 0)),
                   pl.BlockSpec((B, 1), lambda j: (0, 0)),
                   pl.BlockSpec((B, 1), lambda j: (0, 0)),
                   pl.BlockSpec((B, 1), lambda j: (0, 0))],
        out_shape=[jax.ShapeDtypeStruct((B, 1), jnp.float32),
                   jax.ShapeDtypeStruct((B, 1), jnp.float32),
                   jax.ShapeDtypeStruct((B, 1), jnp.int32),
                   jax.ShapeDtypeStruct((B, 1), jnp.int32),
                   jax.ShapeDtypeStruct((B, 1), jnp.float32)],
        scratch_shapes=[pltpu.VMEM((B, 1), jnp.float32)],
        compiler_params=pltpu.CompilerParams(
            dimension_semantics=("arbitrary",)),
        interpret=interpret,
    )

    recover = pl.pallas_call(
        _recover_body,
        grid_spec=pltpu.PrefetchScalarGridSpec(
            num_scalar_prefetch=2,
            grid=(1,),
            in_specs=[
                pl.BlockSpec(memory_space=pl.ANY),
                pl.BlockSpec((B, 1), lambda i, blk, ablk: (0, 0)),
                pl.BlockSpec((B, 1), lambda i, blk, ablk: (0, 0)),
                pl.BlockSpec((B, 1), lambda i, blk, ablk: (0, 0)),
                pl.BlockSpec((B, 1), lambda i, blk, ablk: (0, 0)),
                pl.BlockSpec((B, 1), lambda i, blk, ablk: (0, 0)),
                pl.BlockSpec((B, 1), lambda i, blk, ablk: (0, 0)),
                pl.BlockSpec((B, 1), lambda i, blk, ablk: (0, 0)),
            ],
            out_specs=[pl.BlockSpec((B, 1), lambda i, blk, ablk: (0, 0)),
                       pl.BlockSpec((B, 1), lambda i, blk, ablk: (0, 0))],
            scratch_shapes=[pltpu.VMEM((B, CB), jnp.float32),
                            pltpu.VMEM((B, CB), jnp.float32),
                            pltpu.SemaphoreType.DMA],
        ),
        out_shape=[jax.ShapeDtypeStruct((B, 1), jnp.float32),
                   jax.ShapeDtypeStruct((B, 1), jnp.int32)],
        interpret=interpret,
    )

    @jax.jit
    def run(logits, actions):
        a = actions.astype(jnp.int32).reshape(B, 1)
        lp0, m, blk, it, gt = stream(logits, a)
        ablk = a // CB
        lp, mode = recover(blk.reshape(B), ablk.reshape(B), logits, m, a,
                           lp0, blk, ablk, it, gt)
        return lp, mode

    return run


_run = _build()


def kernel(logits, actions):
    return _run(logits, actions)
